# Initial kernel scaffold; baseline (speedup 1.0000x reference)
import jax
import jax.numpy as jnp
from jax.experimental import pallas as pl


def kernel(x, conv1_w, bn1_gamma, bn1_beta, W1, b1, W2, b2, W3, b3, W4, b4, W5, b5, W6, b6):
    raise NotImplementedError("write your pallas kernel here")


# speedup vs baseline: 16.6081x; 16.6081x over previous
import jax
import jax.numpy as jnp
from jax.experimental import pallas as pl

B, N, PDIM = 8, 2048, 3
KNN = 5
CH = 64
RB = 256
NB = N // RB
CNT = float(B * N * KNN)


def _knn_kernel(pt_ref, prow_ref, wt_ref, umax_ref, umin_ref, stats_ref):
    ptsT = pt_ref[0]
    rows = prow_ref[0]
    wT = wt_ref[...]

    sq_all = jnp.sum(ptsT * ptsT, axis=0, keepdims=True)
    sq_row = jnp.sum(rows * rows, axis=1, keepdims=True)
    rbf = rows.astype(jnp.bfloat16).astype(jnp.float32)
    tbf = ptsT.astype(jnp.bfloat16).astype(jnp.float32)
    cross = (rbf[:, 0:1] * tbf[0:1, :]
             + rbf[:, 1:2] * tbf[1:2, :]
             + rbf[:, 2:3] * tbf[2:3, :])
    d2 = sq_row + sq_all - 2.0 * cross

    iota = jax.lax.broadcasted_iota(jnp.int32, (RB, N), 1)
    umax = None
    umin = None
    m1 = jnp.zeros((1, PDIM), jnp.float32)
    m2 = jnp.zeros((PDIM, PDIM), jnp.float32)
    for r in range(KNN):
        mval = jnp.min(d2, axis=1, keepdims=True)
        idx = jnp.min(jnp.where(d2 <= mval, iota, N),
                      axis=1, keepdims=True)
        mask = iota == idx
        maskf = mask.astype(jnp.float32)
        nx = jnp.sum(maskf * ptsT[0:1, :], axis=1, keepdims=True)
        ny = jnp.sum(maskf * ptsT[1:2, :], axis=1, keepdims=True)
        nz = jnp.sum(maskf * ptsT[2:3, :], axis=1, keepdims=True)
        e = jnp.concatenate([nx, ny, nz], axis=1) - rows
        ebf = e.astype(jnp.bfloat16).astype(jnp.float32)
        z = (ebf[:, 0:1] * wT[0:1, :]
             + ebf[:, 1:2] * wT[1:2, :]
             + ebf[:, 2:3] * wT[2:3, :])
        umax = z if r == 0 else jnp.maximum(umax, z)
        umin = z if r == 0 else jnp.minimum(umin, z)
        m1 = m1 + jnp.sum(e, axis=0, keepdims=True)
        m2 = m2 + jax.lax.dot_general(e, e, (((0,), (0,)), ((), ())),
                                      preferred_element_type=jnp.float32)
        d2 = jnp.where(mask, jnp.inf, d2)

    umax_ref[0] = umax
    umin_ref[0] = umin
    blk = jnp.concatenate(
        [jnp.concatenate([m1, jnp.zeros((1, 128 - PDIM), jnp.float32)], axis=1),
         jnp.concatenate([m2, jnp.zeros((PDIM, 128 - PDIM), jnp.float32)], axis=1),
         jnp.zeros((4, 128), jnp.float32)], axis=0)
    first = jnp.logical_and(pl.program_id(0) == 0, pl.program_id(1) == 0)

    @pl.when(first)
    def _():
        stats_ref[...] = blk

    @pl.when(jnp.logical_not(first))
    def _():
        stats_ref[...] = stats_ref[...] + blk


def _head_kernel(umax_ref, umin_ref, stats_ref, wt_ref, g_ref, bta_ref,
                 w1_ref, b1_ref, w2_ref, b2_ref, w3_ref, b3_ref,
                 w4_ref, b4_ref, w5_ref, b5_ref, w6_ref, b6_ref, out_ref):
    wT = wt_ref[...]
    m1 = stats_ref[0:1, 0:PDIM] * (1.0 / CNT)
    m2 = stats_ref[1:1 + PDIM, 0:PDIM] * (1.0 / CNT)
    mean = (m1[:, 0:1] * wT[0:1, :] + m1[:, 1:2] * wT[1:2, :]
            + m1[:, 2:3] * wT[2:3, :])
    t = (m2[:, 0:1] * wT[0:1, :] + m2[:, 1:2] * wT[1:2, :]
         + m2[:, 2:3] * wT[2:3, :])
    ey2 = jnp.sum(wT[0:PDIM, :] * t, axis=0, keepdims=True)
    var = ey2 - mean * mean
    s = g_ref[...] * jax.lax.rsqrt(var + 1e-5)
    c = bta_ref[...] - mean * s

    pre = jnp.where(s >= 0.0, umax_ref[0] * s, umin_ref[0] * s) + c
    h = jnp.where(pre >= 0.0, pre, 0.2 * pre)
    for wref, bref in ((w1_ref, b1_ref), (w2_ref, b2_ref), (w3_ref, b3_ref),
                       (w4_ref, b4_ref), (w5_ref, b5_ref)):
        h = jax.lax.dot_general(h.astype(jnp.bfloat16), wref[...],
                                (((1,), (0,)), ((), ())),
                                preferred_element_type=jnp.float32)
        h = jnp.maximum(h + bref[...], 0.0)
    out = jax.lax.dot_general(h.astype(jnp.bfloat16), w6_ref[...],
                              (((1,), (0,)), ((), ())),
                              preferred_element_type=jnp.float32) + b6_ref[...]
    out_ref[0] = out


def kernel(x, conv1_w, bn1_gamma, bn1_beta, W1, b1, W2, b2, W3, b3,
           W4, b4, W5, b5, W6, b6):
    points = x[:, :, 0:PDIM]
    pt = jnp.transpose(points, (0, 2, 1))
    wbf = conv1_w.astype(jnp.bfloat16).astype(jnp.float32)
    wT = jnp.zeros((8, CH), jnp.float32).at[0:PDIM, :].set(wbf.T)

    umax, umin, stats = pl.pallas_call(
        _knn_kernel,
        grid=(B, NB),
        in_specs=[
            pl.BlockSpec((1, PDIM, N), lambda b, nb: (b, 0, 0)),
            pl.BlockSpec((1, RB, PDIM), lambda b, nb: (b, nb, 0)),
            pl.BlockSpec((8, CH), lambda b, nb: (0, 0)),
        ],
        out_specs=[
            pl.BlockSpec((1, RB, CH), lambda b, nb: (b, nb, 0)),
            pl.BlockSpec((1, RB, CH), lambda b, nb: (b, nb, 0)),
            pl.BlockSpec((8, 128), lambda b, nb: (0, 0)),
        ],
        out_shape=[
            jax.ShapeDtypeStruct((B, N, CH), jnp.float32),
            jax.ShapeDtypeStruct((B, N, CH), jnp.float32),
            jax.ShapeDtypeStruct((8, 128), jnp.float32),
        ],
    )(pt, points, wT)

    dims = [(CH, 64), (64, 128), (128, 256), (256, 128), (128, 64), (64, 13)]
    ws = [W1.T.astype(jnp.bfloat16), W2.T.astype(jnp.bfloat16),
          W3.T.astype(jnp.bfloat16), W4.T.astype(jnp.bfloat16),
          W5.T.astype(jnp.bfloat16), W6.T.astype(jnp.bfloat16)]
    bs = [b1.reshape(1, -1), b2.reshape(1, -1), b3.reshape(1, -1),
          b4.reshape(1, -1), b5.reshape(1, -1), b6.reshape(1, -1)]

    in_specs = [
        pl.BlockSpec((1, N, CH), lambda b: (b, 0, 0)),
        pl.BlockSpec((1, N, CH), lambda b: (b, 0, 0)),
        pl.BlockSpec((8, 128), lambda b: (0, 0)),
        pl.BlockSpec((8, CH), lambda b: (0, 0)),
        pl.BlockSpec((1, CH), lambda b: (0, 0)),
        pl.BlockSpec((1, CH), lambda b: (0, 0)),
    ]
    operands = [umax, umin, stats, wT,
                bn1_gamma.reshape(1, -1), bn1_beta.reshape(1, -1)]
    for (fi, fo), w, bb in zip(dims, ws, bs):
        in_specs.append(pl.BlockSpec((fi, fo), lambda b: (0, 0)))
        in_specs.append(pl.BlockSpec((1, fo), lambda b: (0, 0)))
        operands.append(w)
        operands.append(bb)

    out = pl.pallas_call(
        _head_kernel,
        grid=(B,),
        in_specs=in_specs,
        out_specs=pl.BlockSpec((1, N, 13), lambda b: (b, 0, 0)),
        out_shape=jax.ShapeDtypeStruct((B, N, 13), jnp.float32),
    )(*operands)
    return out
